# baseline (speedup 1.0000x reference)
import functools

import jax
import jax.numpy as jnp
from jax import lax
from jax.experimental import pallas as pl
from jax.experimental.pallas import tpu as pltpu
from jax.experimental.pallas import tpu_sc as plsc

QP = 64
D = 128
B = 16384
NC = 2
NS = 16
NW = NC * NS
BPW = B // NW
CHUNK = 64
NCH = BPW // CHUNK

_mesh = plsc.VectorSubcoreMesh(core_axis_name="c", subcore_axis_name="s")


@functools.partial(
    pl.kernel,
    mesh=_mesh,
    out_type=jax.ShapeDtypeStruct((NW, BPW, D), jnp.float32),
    scratch_types=[
        pltpu.VMEM((NCH, CHUNK), jnp.int32),
        pltpu.VMEM_SHARED((QP, D), jnp.float32),
        pltpu.VMEM((BPW, D), jnp.float32),
        pltpu.SemaphoreType.DMA,
        pltpu.SemaphoreType.DMA,
        pltpu.SemaphoreType.DMA,
    ],
)
def _sc_gather(idx_hbm, table_hbm, out_hbm, idx_v, table_s, rows_v, sem_t, sem_g, sem_w):
    sid = lax.axis_index("s")
    wid = sid * NC + lax.axis_index("c")

    icopy = pltpu.async_copy(idx_hbm.at[wid], idx_v, sem_g)

    @pl.when(sid == 0)
    def _stage_table():
        pltpu.async_copy(table_hbm, table_s, sem_t).wait()

    icopy.wait()
    plsc.subcore_barrier()

    gathers = [
        pltpu.async_copy(
            table_s.at[idx_v.at[j]],
            rows_v.at[pl.ds(j * CHUNK, CHUNK)],
            sem_g,
        )
        for j in range(NCH)
    ]
    writes = []
    for j in range(NCH):
        gathers[j].wait()
        writes.append(
            pltpu.async_copy(
                rows_v.at[pl.ds(j * CHUNK, CHUNK)],
                out_hbm.at[wid].at[pl.ds(j * CHUNK, CHUNK)],
                sem_w,
            )
        )
    for w in writes:
        w.wait()


def kernel(x, q_scale_enc):
    idx = x.astype(jnp.int32).reshape(NW, NCH, CHUNK)
    table = q_scale_enc.reshape(QP, D)
    out = _sc_gather(idx, table)
    return out.reshape(B, D, 1, 1)


# speedup vs baseline: 1.0096x; 1.0096x over previous
import functools

import jax
import jax.numpy as jnp
from jax import lax
from jax.experimental import pallas as pl
from jax.experimental.pallas import tpu as pltpu
from jax.experimental.pallas import tpu_sc as plsc

QP = 64
D = 128
B = 16384
NC = 2
NS = 16
NW = NC * NS
BPW = B // NW
CHUNK = 128
NCH = BPW // CHUNK

_mesh = plsc.VectorSubcoreMesh(core_axis_name="c", subcore_axis_name="s")


@functools.partial(
    pl.kernel,
    mesh=_mesh,
    out_type=jax.ShapeDtypeStruct((NW, BPW, D), jnp.float32),
    scratch_types=[
        pltpu.VMEM((NCH, CHUNK), jnp.int32),
        pltpu.VMEM_SHARED((2, QP, D), jnp.float32),
        pltpu.VMEM((BPW, D), jnp.float32),
        pltpu.SemaphoreType.DMA,
        pltpu.SemaphoreType.DMA,
        pltpu.SemaphoreType.DMA,
    ],
)
def _sc_gather(idx_hbm, table_hbm, out_hbm, idx_v, table_s, rows_v, sem_t, sem_g, sem_w):
    sid = lax.axis_index("s")
    wid = sid * NC + lax.axis_index("c")

    icopy = pltpu.async_copy(idx_hbm.at[wid], idx_v, sem_g)

    @pl.when(sid < 2)
    def _stage_table():
        pltpu.async_copy(table_hbm, table_s.at[sid % 2], sem_t).wait()

    icopy.wait()
    plsc.subcore_barrier()

    gathers = [
        pltpu.async_copy(
            table_s.at[sid % 2].at[idx_v.at[j]],
            rows_v.at[pl.ds(j * CHUNK, CHUNK)],
            sem_g,
        )
        for j in range(NCH)
    ]
    writes = []
    for j in range(NCH):
        gathers[j].wait()
        writes.append(
            pltpu.async_copy(
                rows_v.at[pl.ds(j * CHUNK, CHUNK)],
                out_hbm.at[wid].at[pl.ds(j * CHUNK, CHUNK)],
                sem_w,
            )
        )
    for w in writes:
        w.wait()


def kernel(x, q_scale_enc):
    idx = x.astype(jnp.int32).reshape(NW, NCH, CHUNK)
    table = q_scale_enc.reshape(QP, D)
    out = _sc_gather(idx, table)
    return out.reshape(B, D, 1, 1)


# speedup vs baseline: 1.0148x; 1.0051x over previous
import functools

import jax
import jax.numpy as jnp
from jax import lax
from jax.experimental import pallas as pl
from jax.experimental.pallas import tpu as pltpu
from jax.experimental.pallas import tpu_sc as plsc

QP = 64
D = 128
B = 16384
NC = 2
NS = 16
NW = NC * NS
BPW = B // NW
CHUNK = 128
NCH = BPW // CHUNK

_mesh = plsc.VectorSubcoreMesh(core_axis_name="c", subcore_axis_name="s")


@functools.partial(
    pl.kernel,
    mesh=_mesh,
    out_type=jax.ShapeDtypeStruct((NW, BPW, D), jnp.float32),
    scratch_types=[
        pltpu.VMEM((NCH, CHUNK), jnp.int32),
        pltpu.VMEM_SHARED((QP, D), jnp.float32),
        pltpu.VMEM((BPW, D), jnp.float32),
        pltpu.SemaphoreType.DMA,
        pltpu.SemaphoreType.DMA,
        pltpu.SemaphoreType.DMA,
    ],
)
def _sc_gather(idx_hbm, table_hbm, out_hbm, idx_v, table_s, rows_v, sem_t, sem_g, sem_w):
    sid = lax.axis_index("s")
    wid = sid * NC + lax.axis_index("c")

    icopy = pltpu.async_copy(idx_hbm.at[wid], idx_v, sem_g)

    @pl.when(sid == 0)
    def _stage_table():
        pltpu.async_copy(table_hbm, table_s, sem_t).wait()

    icopy.wait()
    plsc.subcore_barrier()

    gathers = [
        pltpu.async_copy(
            table_s.at[idx_v.at[j]],
            rows_v.at[pl.ds(j * CHUNK, CHUNK)],
            sem_g,
        )
        for j in range(NCH)
    ]
    writes = []
    for j in range(NCH):
        gathers[j].wait()
        writes.append(
            pltpu.async_copy(
                rows_v.at[pl.ds(j * CHUNK, CHUNK)],
                out_hbm.at[wid].at[pl.ds(j * CHUNK, CHUNK)],
                sem_w,
            )
        )
    for w in writes:
        w.wait()


def kernel(x, q_scale_enc):
    idx = x.astype(jnp.int32).reshape(NW, NCH, CHUNK)
    table = q_scale_enc.reshape(QP, D)
    out = _sc_gather(idx, table)
    return out.reshape(B, D, 1, 1)
